# baseline (speedup 1.0000x reference)
import functools

import jax
import jax.numpy as jnp
from jax import lax
from jax.experimental import pallas as pl
from jax.experimental.pallas import tpu as pltpu
from jax.experimental.pallas import tpu_sc as plsc

_B = 16384
_EMB = 64
_NC = 2
_NS = 16
_NW = _NC * _NS
_BPW = _B // _NW

_mesh = plsc.VectorSubcoreMesh(core_axis_name="c", subcore_axis_name="s")


@functools.partial(
    pl.kernel,
    mesh=_mesh,
    out_type=[
        jax.ShapeDtypeStruct((_B, _EMB), jnp.float32),
        jax.ShapeDtypeStruct((_B, _EMB), jnp.float32),
    ],
    scratch_types=[
        pltpu.VMEM((_BPW,), jnp.int32),
        pltpu.VMEM((_BPW,), jnp.int32),
        pltpu.SemaphoreType.DMA,
        pltpu.SemaphoreType.DMA,
        pltpu.SemaphoreType.DMA,
    ],
)
def _sc_gather(uids, iids, utab, itab, u_out, i_out,
               uids_v, iids_v, sem_i, sem_u, sem_it):
    wid = lax.axis_index("s") * _NC + lax.axis_index("c")
    base = wid * _BPW
    cu = pltpu.async_copy(uids.at[pl.ds(base, _BPW)], uids_v, sem_i)
    ci = pltpu.async_copy(iids.at[pl.ds(base, _BPW)], iids_v, sem_i)
    cu.wait()
    ci.wait()

    def body(g, carry):
        row = g * 16
        vu = uids_v[pl.ds(row, 16)]
        vi = iids_v[pl.ds(row, 16)]
        for j in range(16):
            pltpu.async_copy(utab.at[pl.ds(vu[j], 1)],
                             u_out.at[pl.ds(base + row + j, 1)], sem_u)
            pltpu.async_copy(itab.at[pl.ds(vi[j], 1)],
                             i_out.at[pl.ds(base + row + j, 1)], sem_it)
        return carry

    lax.fori_loop(0, _BPW // 16, body, 0)
    pltpu.make_async_copy(u_out.at[pl.ds(0, _BPW)],
                          u_out.at[pl.ds(base, _BPW)], sem_u).wait()
    pltpu.make_async_copy(i_out.at[pl.ds(0, _BPW)],
                          i_out.at[pl.ds(base, _BPW)], sem_it).wait()


_BLK = 1024


def _mlp_body(u_ref, i_ref, w1u_ref, w1i_ref, b1_ref, w2_ref, b2_ref,
              w3_ref, b3_ref, o_ref):
    hp = lax.Precision.HIGHEST
    acc = jnp.dot(u_ref[...], w1u_ref[...], precision=hp,
                  preferred_element_type=jnp.float32)
    acc = acc + jnp.dot(i_ref[...], w1i_ref[...], precision=hp,
                        preferred_element_type=jnp.float32)
    h1 = jnp.maximum(acc + b1_ref[...], 0.0)
    h2 = jnp.maximum(
        jnp.dot(h1, w2_ref[...], precision=hp,
                preferred_element_type=jnp.float32) + b2_ref[...], 0.0)
    z = jnp.dot(h2, w3_ref[...], precision=hp,
                preferred_element_type=jnp.float32) + b3_ref[...]
    o_ref[...] = jax.nn.sigmoid(z)


def _mlp(u, i, W1u, W1i, b1, W2, b2, W3, b3):
    nblk = _B // _BLK
    full = lambda shape: pl.BlockSpec(shape, lambda j: (0, 0))
    return pl.pallas_call(
        _mlp_body,
        grid=(nblk,),
        in_specs=[
            pl.BlockSpec((_BLK, _EMB), lambda j: (j, 0)),
            pl.BlockSpec((_BLK, _EMB), lambda j: (j, 0)),
            full(W1u.shape),
            full(W1i.shape),
            full(b1.shape),
            full(W2.shape),
            full(b2.shape),
            full(W3.shape),
            full(b3.shape),
        ],
        out_specs=pl.BlockSpec((_BLK, 1), lambda j: (j, 0)),
        out_shape=jax.ShapeDtypeStruct((_B, 1), jnp.float32),
    )(u, i, W1u, W1i, b1, W2, b2, W3, b3)


def kernel(user_ids, item_ids, user_table, item_table, W1, b1, W2, b2, W3, b3):
    u, i = _sc_gather(user_ids, item_ids, user_table, item_table)
    out = _mlp(u, i, W1[:_EMB], W1[_EMB:], b1.reshape(1, -1),
               W2, b2.reshape(1, -1), W3, b3.reshape(1, 1))
    return out[:, 0]


# speedup vs baseline: 1.0977x; 1.0977x over previous
import functools

import jax
import jax.numpy as jnp
from jax import lax
from jax.experimental import pallas as pl
from jax.experimental.pallas import tpu as pltpu
from jax.experimental.pallas import tpu_sc as plsc

_B = 16384
_EMB = 64
_NC = 2
_NS = 16
_NW = _NC * _NS
_BPW = _B // _NW
_CHUNK = 128
_NCHUNK = _BPW // _CHUNK

_mesh = plsc.VectorSubcoreMesh(core_axis_name="c", subcore_axis_name="s")


@functools.partial(
    pl.kernel,
    mesh=_mesh,
    out_type=jax.ShapeDtypeStruct((_B, 2 * _EMB), jnp.float32),
    scratch_types=[
        pltpu.VMEM((_NCHUNK, _CHUNK), jnp.int32),
        pltpu.VMEM((_NCHUNK, _CHUNK), jnp.int32),
        pltpu.VMEM((_BPW, _EMB), jnp.float32),
        pltpu.VMEM((_BPW, _EMB), jnp.float32),
        pltpu.SemaphoreType.DMA,
        pltpu.SemaphoreType.DMA,
    ],
    compiler_params=pltpu.CompilerParams(
        use_tc_tiling_on_sc=False,
        needs_layout_passes=False,
        disable_bounds_checks=True,
    ),
)
def _sc_gather(uids, iids, utab, itab, x_out,
               uidx_v, iidx_v, urows_v, irows_v, sem_u, sem_i):
    wid = lax.axis_index("s") * _NC + lax.axis_index("c")
    pltpu.sync_copy(uids.at[pl.ds(wid * _NCHUNK, _NCHUNK)], uidx_v)
    pltpu.sync_copy(iids.at[pl.ds(wid * _NCHUNK, _NCHUNK)], iidx_v)
    copies = []
    for j in range(_NCHUNK):
        copies.append(pltpu.async_copy(
            utab.at[uidx_v.at[j]],
            urows_v.at[pl.ds(j * _CHUNK, _CHUNK)], sem_u))
        copies.append(pltpu.async_copy(
            itab.at[iidx_v.at[j]],
            irows_v.at[pl.ds(j * _CHUNK, _CHUNK)], sem_i))
    for c in copies:
        c.wait()
    base = wid * _BPW
    pltpu.sync_copy(urows_v, x_out.at[pl.ds(base, _BPW), pl.ds(0, _EMB)])
    pltpu.sync_copy(irows_v,
                    x_out.at[pl.ds(base, _BPW), pl.ds(_EMB, _EMB)])


_BLK = 1024


def _mlp_body(x_ref, w1_ref, b1_ref, w2_ref, b2_ref, w3_ref, b3_ref, o_ref):
    hp = lax.Precision.HIGHEST
    h1 = jnp.maximum(
        jnp.dot(x_ref[...], w1_ref[...], precision=hp,
                preferred_element_type=jnp.float32) + b1_ref[...], 0.0)
    h2 = jnp.maximum(
        jnp.dot(h1, w2_ref[...], precision=hp,
                preferred_element_type=jnp.float32) + b2_ref[...], 0.0)
    z = jnp.dot(h2, w3_ref[...], precision=hp,
                preferred_element_type=jnp.float32) + b3_ref[...]
    o_ref[...] = jax.nn.sigmoid(z)


def _mlp(x, W1, b1, W2, b2, W3, b3):
    nblk = _B // _BLK
    full = lambda shape: pl.BlockSpec(shape, lambda j: (0, 0))
    return pl.pallas_call(
        _mlp_body,
        grid=(nblk,),
        in_specs=[
            pl.BlockSpec((_BLK, 2 * _EMB), lambda j: (j, 0)),
            full(W1.shape),
            full(b1.shape),
            full(W2.shape),
            full(b2.shape),
            full(W3.shape),
            full(b3.shape),
        ],
        out_specs=pl.BlockSpec((_BLK, 1), lambda j: (j, 0)),
        out_shape=jax.ShapeDtypeStruct((_B, 1), jnp.float32),
    )(x, W1, b1, W2, b2, W3, b3)


def kernel(user_ids, item_ids, user_table, item_table, W1, b1, W2, b2, W3, b3):
    uids2 = user_ids.reshape(_B // _CHUNK, _CHUNK)
    iids2 = item_ids.reshape(_B // _CHUNK, _CHUNK)
    x = _sc_gather(uids2, iids2, user_table, item_table)
    out = _mlp(x, W1, b1.reshape(1, -1), W2, b2.reshape(1, -1),
               W3, b3.reshape(1, 1))
    return out[:, 0]
